# baseline (speedup 1.0000x reference)
import functools

import jax
import jax.numpy as jnp
from jax.experimental import pallas as pl


def _concat_block_kernel(cs_ref, x_ref, out_ref, *, block_rows):
    i = pl.program_id(0)
    j = (jax.lax.broadcasted_iota(jnp.int32, (block_rows, 1), 0)
         + (i * block_rows)).astype(jnp.float32)
    cs = cs_ref[0, :][None, :]
    le = cs <= j
    start = jnp.max(jnp.where(le, cs, 0.0), axis=1, keepdims=True)
    nxt = jnp.min(jnp.where(le, jnp.inf, cs), axis=1, keepdims=True)
    pe = (j - start + 1.0) / (nxt - start)
    out_ref[:, :] = jnp.concatenate([pe, x_ref[:, :]], axis=1)


@jax.jit
def kernel(x, lengths):
    n, d = x.shape
    s = lengths.shape[0]
    block_rows = 64
    grid = n // block_rows
    cs = jnp.cumsum(lengths.astype(jnp.float32)).reshape(1, s)
    return pl.pallas_call(
        functools.partial(_concat_block_kernel, block_rows=block_rows),
        grid=(grid,),
        in_specs=[
            pl.BlockSpec((1, s), lambda i: (0, 0)),
            pl.BlockSpec((block_rows, d), lambda i: (i, 0)),
        ],
        out_specs=pl.BlockSpec((block_rows, d + 1), lambda i: (i, 0)),
        out_shape=jax.ShapeDtypeStruct((n, d + 1), x.dtype),
    )(cs, x)


# speedup vs baseline: 1.3983x; 1.3983x over previous
import functools

import jax
import jax.numpy as jnp
from jax import lax
from jax.experimental import pallas as pl
from jax.experimental.pallas import tpu as pltpu
from jax.experimental.pallas import tpu_sc as plsc

_LANES = 16
_WORKERS = 32


def _sc_pe_kernel(cs_hbm, pe_hbm, cs_v, buf_v, *, num_seg, num_groups):
    wid = lax.axis_index("s") * 2 + lax.axis_index("c")
    pltpu.sync_copy(cs_hbm, cs_v)

    iota = lax.iota(jnp.int32, _LANES)
    groups_per_worker = (num_groups + _WORKERS - 1) // _WORKERS
    for t in range(groups_per_worker):
        g = wid + _WORKERS * t

        @pl.when(g < num_groups)
        def _(g=g):
            j = (g * _LANES + iota).astype(jnp.float32)
            pos = jnp.zeros((_LANES,), jnp.int32)
            bit = num_seg // 2
            while bit:
                cand = pos + bit
                val = plsc.load_gather(cs_v, [cand - 1])
                pos = jnp.where(val <= j, cand, pos)
                bit //= 2
            start = jnp.where(
                pos == 0, jnp.float32(0.0),
                plsc.load_gather(cs_v, [jnp.maximum(pos - 1, 0)]))
            nxt = plsc.load_gather(cs_v, [pos])
            buf_v[...] = (j - start + 1.0) / (nxt - start)
            pltpu.sync_copy(buf_v, pe_hbm.at[pl.ds(g * _LANES, _LANES)])


def _tc_concat_kernel(pe_ref, x_ref, out_ref):
    out_ref[:, :] = jnp.concatenate([pe_ref[:, :], x_ref[:, :]], axis=1)


@jax.jit
def kernel(x, lengths):
    n, d = x.shape
    s = lengths.shape[0]

    sc_pe = pl.kernel(
        functools.partial(_sc_pe_kernel, num_seg=s, num_groups=n // _LANES),
        out_type=jax.ShapeDtypeStruct((n,), jnp.float32),
        mesh=plsc.VectorSubcoreMesh(core_axis_name="c", subcore_axis_name="s"),
        compiler_params=pltpu.CompilerParams(needs_layout_passes=False),
        scratch_types=[
            pltpu.VMEM((s,), jnp.float32),
            pltpu.VMEM((_LANES,), jnp.float32),
        ],
    )
    pe = sc_pe(jnp.cumsum(lengths.astype(jnp.float32)))

    block_rows = 2032
    return pl.pallas_call(
        _tc_concat_kernel,
        grid=(n // block_rows,),
        in_specs=[
            pl.BlockSpec((block_rows, 1), lambda i: (i, 0)),
            pl.BlockSpec((block_rows, d), lambda i: (i, 0)),
        ],
        out_specs=pl.BlockSpec((block_rows, d + 1), lambda i: (i, 0)),
        out_shape=jax.ShapeDtypeStruct((n, d + 1), x.dtype),
    )(pe.reshape(n, 1), x)


# speedup vs baseline: 1.4181x; 1.0141x over previous
import functools

import jax
import jax.numpy as jnp
from jax import lax
from jax.experimental import pallas as pl
from jax.experimental.pallas import tpu as pltpu
from jax.experimental.pallas import tpu_sc as plsc

_LANES = 16
_WORKERS = 32


def _sc_pe_kernel(cs_hbm, pe_hbm, cs_v, buf_v, *, num_seg, groups_per_worker):
    wid = lax.axis_index("s") * 2 + lax.axis_index("c")
    pltpu.sync_copy(cs_hbm, cs_v)

    iota = lax.iota(jnp.int32, _LANES)
    base = wid * (groups_per_worker * _LANES)
    for t in range(groups_per_worker):
        j = (base + t * _LANES + iota).astype(jnp.float32)
        pos = jnp.zeros((_LANES,), jnp.int32)
        bit = num_seg // 2
        while bit:
            cand = pos + bit
            val = plsc.load_gather(cs_v, [cand - 1])
            pos = jnp.where(val <= j, cand, pos)
            bit //= 2
        start = jnp.where(
            pos == 0, jnp.float32(0.0),
            plsc.load_gather(cs_v, [jnp.maximum(pos - 1, 0)]))
        nxt = plsc.load_gather(cs_v, [pos])
        buf_v[pl.ds(t * _LANES, _LANES)] = (j - start + 1.0) / (nxt - start)

    pltpu.sync_copy(buf_v, pe_hbm.at[pl.ds(base, groups_per_worker * _LANES)])


def _tc_concat_kernel(pe_ref, x_ref, out_ref):
    out_ref[:, :] = jnp.concatenate([pe_ref[:, :], x_ref[:, :]], axis=1)


@jax.jit
def kernel(x, lengths):
    n, d = x.shape
    s = lengths.shape[0]

    n_pad = -(-n // (_WORKERS * _LANES)) * (_WORKERS * _LANES)
    gpw = n_pad // (_WORKERS * _LANES)
    sc_pe = pl.kernel(
        functools.partial(_sc_pe_kernel, num_seg=s, groups_per_worker=gpw),
        out_type=jax.ShapeDtypeStruct((n_pad,), jnp.float32),
        mesh=plsc.VectorSubcoreMesh(core_axis_name="c", subcore_axis_name="s"),
        compiler_params=pltpu.CompilerParams(needs_layout_passes=False),
        scratch_types=[
            pltpu.VMEM((s,), jnp.float32),
            pltpu.VMEM((gpw * _LANES,), jnp.float32),
        ],
    )
    pe = sc_pe(jnp.cumsum(lengths.astype(jnp.float32)))

    block_rows = 2032
    return pl.pallas_call(
        _tc_concat_kernel,
        grid=(n // block_rows,),
        in_specs=[
            pl.BlockSpec((block_rows, 1), lambda i: (i, 0)),
            pl.BlockSpec((block_rows, d), lambda i: (i, 0)),
        ],
        out_specs=pl.BlockSpec((block_rows, d + 1), lambda i: (i, 0)),
        out_shape=jax.ShapeDtypeStruct((n, d + 1), x.dtype),
    )(pe[:n].reshape(n, 1), x)
